# baseline (speedup 1.0000x reference)
import functools

import jax
import jax.numpy as jnp
from jax import lax
from jax.experimental import pallas as pl
from jax.experimental.pallas import tpu as pltpu
from jax.experimental.pallas import tpu_sc as plsc

HIDDEN = 1024
FFN = 2048
NE = 8
T = 2048
TOPK = 2
TM = 128
F_TILE = 512
NF = FFN // F_TILE
FH = F_TILE // 2
NPAD = T * TOPK + NE * TM
NT = NPAD // TM
NW = 32
TPW = T // NW
CH = 256


def _router_body(x_ref, gw_ref, logits_ref, e0_ref, e1_ref, w0_ref, w1_ref,
                 r0_ref, r1_ref, cnt_ref, h_ref):
    x = x_ref[...]
    logits = lax.dot_general(x, gw_ref[...], (((1,), (1,)), ((), ())),
                             preferred_element_type=jnp.float32)
    logits_ref[...] = logits
    col = lax.broadcasted_iota(jnp.int32, logits.shape, 1)
    m1 = jnp.max(logits, axis=1, keepdims=True)
    e0 = jnp.min(jnp.where(logits == m1, col, NE), axis=1, keepdims=True)
    masked = jnp.where(col == e0, jnp.float32(-1e30), logits)
    m2 = jnp.max(masked, axis=1, keepdims=True)
    e1 = jnp.min(jnp.where(masked == m2, col, NE), axis=1, keepdims=True)
    s = jnp.exp(m2 - m1)
    denom = 1.0 + s
    e0_ref[...] = e0
    e1_ref[...] = e1
    w0_ref[...] = 1.0 / denom
    w1_ref[...] = s / denom
    h_ref[...] = ((col == e0) | (col == e1)).astype(jnp.float32)

    ri = lax.broadcasted_iota(jnp.int32, (CH, CH), 0)
    ci = lax.broadcasted_iota(jnp.int32, (CH, CH), 1)
    tri = (ri > ci).astype(jnp.bfloat16)

    def chunk(i, carry):
        sl = pl.ds(i * CH, CH)
        hc = h_ref[sl, :]
        cc = lax.dot_general(tri, hc.astype(jnp.bfloat16),
                             (((1,), (0,)), ((), ())),
                             preferred_element_type=jnp.float32) + carry
        colc = lax.broadcasted_iota(jnp.int32, (CH, NE), 1)
        e0c = e0_ref[sl, :]
        e1c = e1_ref[sl, :]
        zero = jnp.float32(0.0)
        r0_ref[sl, :] = jnp.sum(jnp.where(colc == e0c, cc, zero), axis=1,
                                keepdims=True).astype(jnp.int32)
        r1_ref[sl, :] = jnp.sum(jnp.where(colc == e1c, cc, zero), axis=1,
                                keepdims=True).astype(jnp.int32)
        return carry + jnp.sum(hc, axis=0, keepdims=True)

    cnt_ref[...] = lax.fori_loop(0, T // CH, chunk,
                                 jnp.zeros((1, NE), jnp.float32))


def _grouped_body(offs_ref, nt_ref, xs_ref,
                  w1a_ref, w1b_ref, w3a_ref, w3b_ref, w2a_ref, w2b_ref,
                  out_ref, c1a_ref, c1b_ref, c3a_ref, c3b_ref, c2a_ref,
                  c2b_ref):
    e = pl.program_id(0)
    f = pl.program_id(1)
    c1a_ref[...] = w1a_ref[0].astype(jnp.bfloat16)
    c1b_ref[...] = w1b_ref[0].astype(jnp.bfloat16)
    c3a_ref[...] = w3a_ref[0].astype(jnp.bfloat16)
    c3b_ref[...] = w3b_ref[0].astype(jnp.bfloat16)
    c2a_ref[...] = w2a_ref[0].astype(jnp.bfloat16)
    c2b_ref[...] = w2b_ref[0].astype(jnp.bfloat16)
    base = offs_ref[e]
    ntile = nt_ref[e]

    def tile(t, carry):
        sl = pl.ds(pl.multiple_of(base + t * TM, TM), TM)
        xb = xs_ref[sl, :].astype(jnp.bfloat16)
        nt_dims = (((1,), (1,)), ((), ()))
        y1a = lax.dot_general(xb, c1a_ref[...], nt_dims,
                              preferred_element_type=jnp.float32)
        y3a = lax.dot_general(xb, c3a_ref[...], nt_dims,
                              preferred_element_type=jnp.float32)
        ha = ((y1a * lax.logistic(y1a)) * y3a).astype(jnp.bfloat16)
        y1b = lax.dot_general(xb, c1b_ref[...], nt_dims,
                              preferred_element_type=jnp.float32)
        y3b = lax.dot_general(xb, c3b_ref[...], nt_dims,
                              preferred_element_type=jnp.float32)
        hb = ((y1b * lax.logistic(y1b)) * y3b).astype(jnp.bfloat16)
        yp = (lax.dot_general(ha, c2a_ref[...], nt_dims,
                              preferred_element_type=jnp.float32)
              + lax.dot_general(hb, c2b_ref[...], nt_dims,
                                preferred_element_type=jnp.float32))

        @pl.when(f == 0)
        def _set():
            out_ref[sl, :] = yp

        @pl.when(f != 0)
        def _acc():
            out_ref[sl, :] += yp

        return carry

    lax.fori_loop(0, ntile, tile, 0)


def _combine_body(a_ref, b_ref, w0_ref, w1_ref, o_ref):
    o_ref[...] = a_ref[...] * w0_ref[...] + b_ref[...] * w1_ref[...]


def _sc_mesh():
    return plsc.VectorSubcoreMesh(core_axis_name="c", subcore_axis_name="s")


def _dispatch_scatter(x_f32, pos0, pos1):
    @functools.partial(
        pl.kernel, mesh=_sc_mesh(),
        out_type=jax.ShapeDtypeStruct((NPAD, HIDDEN), jnp.float32),
        scratch_types=[
            pltpu.VMEM((TPW,), jnp.int32),
            pltpu.VMEM((TPW,), jnp.int32),
            pltpu.VMEM((TPW, HIDDEN), jnp.float32),
            pltpu.SemaphoreType.DMA,
        ],
    )
    def k(x_hbm, p0_hbm, p1_hbm, xs_hbm, i0_v, i1_v, rows_v, sem):
        wid = lax.axis_index("s") * 2 + lax.axis_index("c")
        base = wid * TPW
        pltpu.sync_copy(p0_hbm.at[pl.ds(base, TPW)], i0_v)
        pltpu.sync_copy(p1_hbm.at[pl.ds(base, TPW)], i1_v)
        pltpu.sync_copy(x_hbm.at[pl.ds(base, TPW)], rows_v)
        pltpu.async_copy(rows_v, xs_hbm.at[i0_v], sem).wait()
        pltpu.async_copy(rows_v, xs_hbm.at[i1_v], sem).wait()

    return k(x_f32, pos0, pos1)


def _combine_gather(ys, pos0, pos1):
    @functools.partial(
        pl.kernel, mesh=_sc_mesh(),
        out_type=(jax.ShapeDtypeStruct((T, HIDDEN), jnp.float32),
                  jax.ShapeDtypeStruct((T, HIDDEN), jnp.float32)),
        scratch_types=[
            pltpu.VMEM((TPW,), jnp.int32),
            pltpu.VMEM((TPW,), jnp.int32),
            pltpu.VMEM((TPW, HIDDEN), jnp.float32),
            pltpu.SemaphoreType.DMA,
        ],
    )
    def k(ys_hbm, p0_hbm, p1_hbm, a_hbm, b_hbm, i0_v, i1_v, rows_v, sem):
        wid = lax.axis_index("s") * 2 + lax.axis_index("c")
        base = wid * TPW
        pltpu.sync_copy(p0_hbm.at[pl.ds(base, TPW)], i0_v)
        pltpu.sync_copy(p1_hbm.at[pl.ds(base, TPW)], i1_v)
        pltpu.async_copy(ys_hbm.at[i0_v], rows_v, sem).wait()
        pltpu.sync_copy(rows_v, a_hbm.at[pl.ds(base, TPW)])
        pltpu.async_copy(ys_hbm.at[i1_v], rows_v, sem).wait()
        pltpu.sync_copy(rows_v, b_hbm.at[pl.ds(base, TPW)])

    return k(ys, pos0, pos1)


def kernel(hidden_states, gate_w, w1, w3, w2):
    b, s, hd = hidden_states.shape
    x2 = hidden_states.reshape(T, HIDDEN)

    (logits, e0, e1, wt0, wt1, r0, r1, cnt) = pl.pallas_call(
        _router_body,
        out_shape=(
            jax.ShapeDtypeStruct((T, NE), jnp.float32),
            jax.ShapeDtypeStruct((T, 1), jnp.int32),
            jax.ShapeDtypeStruct((T, 1), jnp.int32),
            jax.ShapeDtypeStruct((T, 1), jnp.float32),
            jax.ShapeDtypeStruct((T, 1), jnp.float32),
            jax.ShapeDtypeStruct((T, 1), jnp.int32),
            jax.ShapeDtypeStruct((T, 1), jnp.int32),
            jax.ShapeDtypeStruct((1, NE), jnp.float32),
        ),
        scratch_shapes=[pltpu.VMEM((T, NE), jnp.float32)],
    )(x2, gate_w)

    counts = cnt.reshape(NE).astype(jnp.int32)
    padded = ((counts + TM - 1) // TM) * TM
    offs = jnp.concatenate([jnp.zeros((1,), jnp.int32),
                            jnp.cumsum(padded)[:-1]])
    pos0 = (jnp.take(offs, e0.reshape(T)) + r0.reshape(T)).astype(jnp.int32)
    pos1 = (jnp.take(offs, e1.reshape(T)) + r1.reshape(T)).astype(jnp.int32)
    ntiles = (padded // TM).astype(jnp.int32)

    xs = _dispatch_scatter(x2, pos0, pos1)

    ys = pl.pallas_call(
        _grouped_body,
        grid_spec=pltpu.PrefetchScalarGridSpec(
            num_scalar_prefetch=2,
            grid=(NE, NF),
            in_specs=[
                pl.BlockSpec((NPAD, HIDDEN), lambda e, f, offs, nt: (0, 0)),
                pl.BlockSpec((1, FH, HIDDEN),
                             lambda e, f, offs, nt: (e, 2 * f, 0)),
                pl.BlockSpec((1, FH, HIDDEN),
                             lambda e, f, offs, nt: (e, 2 * f + 1, 0)),
                pl.BlockSpec((1, FH, HIDDEN),
                             lambda e, f, offs, nt: (e, 2 * f, 0)),
                pl.BlockSpec((1, FH, HIDDEN),
                             lambda e, f, offs, nt: (e, 2 * f + 1, 0)),
                pl.BlockSpec((1, HIDDEN, FH),
                             lambda e, f, offs, nt: (e, 0, 2 * f)),
                pl.BlockSpec((1, HIDDEN, FH),
                             lambda e, f, offs, nt: (e, 0, 2 * f + 1)),
            ],
            out_specs=pl.BlockSpec((NPAD, HIDDEN),
                                   lambda e, f, offs, nt: (0, 0)),
            scratch_shapes=[
                pltpu.VMEM((FH, HIDDEN), jnp.bfloat16),
                pltpu.VMEM((FH, HIDDEN), jnp.bfloat16),
                pltpu.VMEM((FH, HIDDEN), jnp.bfloat16),
                pltpu.VMEM((FH, HIDDEN), jnp.bfloat16),
                pltpu.VMEM((HIDDEN, FH), jnp.bfloat16),
                pltpu.VMEM((HIDDEN, FH), jnp.bfloat16),
            ],
        ),
        out_shape=jax.ShapeDtypeStruct((NPAD, HIDDEN), jnp.float32),
    )(offs, ntiles, xs, w1, w1, w3, w3, w2, w2)

    a, bb = _combine_gather(ys, pos0, pos1)

    out = pl.pallas_call(
        _combine_body,
        out_shape=jax.ShapeDtypeStruct((T, HIDDEN), jnp.float32),
    )(a, bb, wt0, wt1)

    return out.reshape(b, s, hd), logits


# speedup vs baseline: 1.0821x; 1.0821x over previous
import functools

import jax
import jax.numpy as jnp
from jax import lax
from jax.experimental import pallas as pl
from jax.experimental.pallas import tpu as pltpu
from jax.experimental.pallas import tpu_sc as plsc

HIDDEN = 1024
FFN = 2048
NE = 8
T = 2048
TOPK = 2
TM = 256
F_TILE = 512
NF = FFN // F_TILE
NPAD = T * TOPK + NE * TM
NT = NPAD // TM
NW = 32
TPW = T // NW
CH = 256


def _router_body(x_ref, gw_ref, logits_ref, e0_ref, e1_ref, w0_ref, w1_ref,
                 r0_ref, r1_ref, cnt_ref, h_ref):
    x = x_ref[...]
    logits = lax.dot_general(x, gw_ref[...], (((1,), (1,)), ((), ())),
                             preferred_element_type=jnp.float32)
    logits_ref[...] = logits
    col = lax.broadcasted_iota(jnp.int32, logits.shape, 1)
    m1 = jnp.max(logits, axis=1, keepdims=True)
    e0 = jnp.min(jnp.where(logits == m1, col, NE), axis=1, keepdims=True)
    masked = jnp.where(col == e0, jnp.float32(-1e30), logits)
    m2 = jnp.max(masked, axis=1, keepdims=True)
    e1 = jnp.min(jnp.where(masked == m2, col, NE), axis=1, keepdims=True)
    s = jnp.exp(m2 - m1)
    denom = 1.0 + s
    e0_ref[...] = e0
    e1_ref[...] = e1
    w0_ref[...] = 1.0 / denom
    w1_ref[...] = s / denom
    h_ref[...] = ((col == e0) | (col == e1)).astype(jnp.float32)

    ri = lax.broadcasted_iota(jnp.int32, (CH, CH), 0)
    ci = lax.broadcasted_iota(jnp.int32, (CH, CH), 1)
    tri = (ri > ci).astype(jnp.bfloat16)

    def chunk(i, carry):
        sl = pl.ds(i * CH, CH)
        hc = h_ref[sl, :]
        cc = lax.dot_general(tri, hc.astype(jnp.bfloat16),
                             (((1,), (0,)), ((), ())),
                             preferred_element_type=jnp.float32) + carry
        colc = lax.broadcasted_iota(jnp.int32, (CH, NE), 1)
        e0c = e0_ref[sl, :]
        e1c = e1_ref[sl, :]
        zero = jnp.float32(0.0)
        r0_ref[sl, :] = jnp.sum(jnp.where(colc == e0c, cc, zero), axis=1,
                                keepdims=True).astype(jnp.int32)
        r1_ref[sl, :] = jnp.sum(jnp.where(colc == e1c, cc, zero), axis=1,
                                keepdims=True).astype(jnp.int32)
        return carry + jnp.sum(hc, axis=0, keepdims=True)

    cnt_ref[...] = lax.fori_loop(0, T // CH, chunk,
                                 jnp.zeros((1, NE), jnp.float32))


def _grouped_body(te_ref, xs_ref, w1_ref, w3_ref, w2_ref, out_ref,
                  xsb_ref, w1b_ref, w3b_ref, w2b_ref):
    f = pl.program_id(0)
    i = pl.program_id(1)
    sl = pl.ds(i * TM, TM)
    prev = te_ref[jnp.maximum(i - 1, 0)]
    changed = (i == 0) | (te_ref[i] != prev)

    @pl.when(changed)
    def _cast():
        w1b_ref[...] = w1_ref[0].astype(jnp.bfloat16)
        w3b_ref[...] = w3_ref[0].astype(jnp.bfloat16)
        w2b_ref[...] = w2_ref[0].astype(jnp.bfloat16)

    @pl.when(f == 0)
    def _cx():
        xsb_ref[sl, :] = xs_ref[...].astype(jnp.bfloat16)

    xb = xsb_ref[sl, :]
    y1 = lax.dot_general(xb, w1b_ref[...], (((1,), (1,)), ((), ())),
                         preferred_element_type=jnp.float32)
    y3 = lax.dot_general(xb, w3b_ref[...], (((1,), (1,)), ((), ())),
                         preferred_element_type=jnp.float32)
    h = ((y1 * lax.logistic(y1)) * y3).astype(jnp.bfloat16)
    yp = lax.dot_general(h, w2b_ref[...], (((1,), (1,)), ((), ())),
                         preferred_element_type=jnp.float32)

    @pl.when(f == 0)
    def _set():
        out_ref[sl, :] = yp

    @pl.when(f != 0)
    def _acc():
        out_ref[sl, :] += yp


def _combine_body(a_ref, b_ref, w0_ref, w1_ref, o_ref):
    o_ref[...] = a_ref[...] * w0_ref[...] + b_ref[...] * w1_ref[...]


def _sc_mesh():
    return plsc.VectorSubcoreMesh(core_axis_name="c", subcore_axis_name="s")


def _dispatch_scatter(x_f32, pos0, pos1):
    @functools.partial(
        pl.kernel, mesh=_sc_mesh(),
        out_type=jax.ShapeDtypeStruct((NPAD, HIDDEN), jnp.float32),
        scratch_types=[
            pltpu.VMEM((TPW,), jnp.int32),
            pltpu.VMEM((TPW,), jnp.int32),
            pltpu.VMEM((TPW, HIDDEN), jnp.float32),
            pltpu.SemaphoreType.DMA,
        ],
    )
    def k(x_hbm, p0_hbm, p1_hbm, xs_hbm, i0_v, i1_v, rows_v, sem):
        wid = lax.axis_index("s") * 2 + lax.axis_index("c")
        base = wid * TPW
        pltpu.sync_copy(p0_hbm.at[pl.ds(base, TPW)], i0_v)
        pltpu.sync_copy(p1_hbm.at[pl.ds(base, TPW)], i1_v)
        pltpu.sync_copy(x_hbm.at[pl.ds(base, TPW)], rows_v)
        pltpu.async_copy(rows_v, xs_hbm.at[i0_v], sem).wait()
        pltpu.async_copy(rows_v, xs_hbm.at[i1_v], sem).wait()

    return k(x_f32, pos0, pos1)


def _combine_gather(ys, pos0, pos1):
    @functools.partial(
        pl.kernel, mesh=_sc_mesh(),
        out_type=(jax.ShapeDtypeStruct((T, HIDDEN), jnp.float32),
                  jax.ShapeDtypeStruct((T, HIDDEN), jnp.float32)),
        scratch_types=[
            pltpu.VMEM((TPW,), jnp.int32),
            pltpu.VMEM((TPW,), jnp.int32),
            pltpu.VMEM((TPW, HIDDEN), jnp.float32),
            pltpu.SemaphoreType.DMA,
        ],
    )
    def k(ys_hbm, p0_hbm, p1_hbm, a_hbm, b_hbm, i0_v, i1_v, rows_v, sem):
        wid = lax.axis_index("s") * 2 + lax.axis_index("c")
        base = wid * TPW
        pltpu.sync_copy(p0_hbm.at[pl.ds(base, TPW)], i0_v)
        pltpu.sync_copy(p1_hbm.at[pl.ds(base, TPW)], i1_v)
        pltpu.async_copy(ys_hbm.at[i0_v], rows_v, sem).wait()
        pltpu.sync_copy(rows_v, a_hbm.at[pl.ds(base, TPW)])
        pltpu.async_copy(ys_hbm.at[i1_v], rows_v, sem).wait()
        pltpu.sync_copy(rows_v, b_hbm.at[pl.ds(base, TPW)])

    return k(ys, pos0, pos1)


def kernel(hidden_states, gate_w, w1, w3, w2):
    b, s, hd = hidden_states.shape
    x2 = hidden_states.reshape(T, HIDDEN)

    (logits, e0, e1, wt0, wt1, r0, r1, cnt) = pl.pallas_call(
        _router_body,
        out_shape=(
            jax.ShapeDtypeStruct((T, NE), jnp.float32),
            jax.ShapeDtypeStruct((T, 1), jnp.int32),
            jax.ShapeDtypeStruct((T, 1), jnp.int32),
            jax.ShapeDtypeStruct((T, 1), jnp.float32),
            jax.ShapeDtypeStruct((T, 1), jnp.float32),
            jax.ShapeDtypeStruct((T, 1), jnp.int32),
            jax.ShapeDtypeStruct((T, 1), jnp.int32),
            jax.ShapeDtypeStruct((1, NE), jnp.float32),
        ),
        scratch_shapes=[pltpu.VMEM((T, NE), jnp.float32)],
    )(x2, gate_w)

    counts = cnt.reshape(NE).astype(jnp.int32)
    padded = ((counts + TM - 1) // TM) * TM
    offs = jnp.concatenate([jnp.zeros((1,), jnp.int32),
                            jnp.cumsum(padded)[:-1]])
    pos0 = (jnp.take(offs, e0.reshape(T)) + r0.reshape(T)).astype(jnp.int32)
    pos1 = (jnp.take(offs, e1.reshape(T)) + r1.reshape(T)).astype(jnp.int32)
    starts = offs // TM
    tile_expert = (jnp.sum(
        (jnp.arange(NT, dtype=jnp.int32)[:, None] >= starts[None, :])
        .astype(jnp.int32), axis=1) - 1).astype(jnp.int32)

    xs = _dispatch_scatter(x2, pos0, pos1)

    ys = pl.pallas_call(
        _grouped_body,
        grid_spec=pltpu.PrefetchScalarGridSpec(
            num_scalar_prefetch=1,
            grid=(NF, NT),
            in_specs=[
                pl.BlockSpec((TM, HIDDEN),
                             lambda f, i, te: (jnp.where(f == 0, i, 0), 0)),
                pl.BlockSpec((1, F_TILE, HIDDEN), lambda f, i, te: (te[i], f, 0)),
                pl.BlockSpec((1, F_TILE, HIDDEN), lambda f, i, te: (te[i], f, 0)),
                pl.BlockSpec((1, HIDDEN, F_TILE), lambda f, i, te: (te[i], 0, f)),
            ],
            out_specs=pl.BlockSpec((NPAD, HIDDEN), lambda f, i, te: (0, 0)),
            scratch_shapes=[
                pltpu.VMEM((NPAD, HIDDEN), jnp.bfloat16),
                pltpu.VMEM((F_TILE, HIDDEN), jnp.bfloat16),
                pltpu.VMEM((F_TILE, HIDDEN), jnp.bfloat16),
                pltpu.VMEM((HIDDEN, F_TILE), jnp.bfloat16),
            ],
        ),
        out_shape=jax.ShapeDtypeStruct((NPAD, HIDDEN), jnp.float32),
    )(tile_expert, xs, w1, w3, w2)

    a, bb = _combine_gather(ys, pos0, pos1)

    out = pl.pallas_call(
        _combine_body,
        out_shape=jax.ShapeDtypeStruct((T, HIDDEN), jnp.float32),
    )(a, bb, wt0, wt1)

    return out.reshape(b, s, hd), logits


# speedup vs baseline: 1.1166x; 1.0319x over previous
import functools

import jax
import jax.numpy as jnp
from jax import lax
from jax.experimental import pallas as pl
from jax.experimental.pallas import tpu as pltpu
from jax.experimental.pallas import tpu_sc as plsc

HIDDEN = 1024
FFN = 2048
NE = 8
T = 2048
TOPK = 2
TM = 256
F_TILE = 512
NF = FFN // F_TILE
NPAD = T * TOPK + NE * TM
NT = NPAD // TM
NW = 32
TPW = T // NW
CH = 256


def _router_body(x_ref, gw_ref, logits_ref, w0_ref, w1_ref,
                 p0_ref, p1_ref, te_ref, e0_ref, e1_ref, r0_ref, r1_ref,
                 h_ref):
    x = x_ref[...]
    logits = lax.dot_general(x, gw_ref[...], (((1,), (1,)), ((), ())),
                             preferred_element_type=jnp.float32)
    logits_ref[...] = logits
    col = lax.broadcasted_iota(jnp.int32, logits.shape, 1)
    m1 = jnp.max(logits, axis=1, keepdims=True)
    e0 = jnp.min(jnp.where(logits == m1, col, NE), axis=1, keepdims=True)
    masked = jnp.where(col == e0, jnp.float32(-1e30), logits)
    m2 = jnp.max(masked, axis=1, keepdims=True)
    e1 = jnp.min(jnp.where(masked == m2, col, NE), axis=1, keepdims=True)
    s = jnp.exp(m2 - m1)
    denom = 1.0 + s
    e0_ref[...] = e0
    e1_ref[...] = e1
    w0_ref[...] = 1.0 / denom
    w1_ref[...] = s / denom
    h_ref[...] = ((col == e0) | (col == e1)).astype(jnp.float32)

    ri = lax.broadcasted_iota(jnp.int32, (CH, CH), 0)
    ci = lax.broadcasted_iota(jnp.int32, (CH, CH), 1)
    tri = (ri > ci).astype(jnp.bfloat16)

    def chunk(i, carry):
        sl = pl.ds(i * CH, CH)
        hc = h_ref[sl, :]
        cc = lax.dot_general(tri, hc.astype(jnp.bfloat16),
                             (((1,), (0,)), ((), ())),
                             preferred_element_type=jnp.float32) + carry
        colc = lax.broadcasted_iota(jnp.int32, (CH, NE), 1)
        e0c = e0_ref[sl, :]
        e1c = e1_ref[sl, :]
        zero = jnp.float32(0.0)
        r0_ref[sl, :] = jnp.sum(jnp.where(colc == e0c, cc, zero), axis=1,
                                keepdims=True)
        r1_ref[sl, :] = jnp.sum(jnp.where(colc == e1c, cc, zero), axis=1,
                                keepdims=True)
        return carry + jnp.sum(hc, axis=0, keepdims=True)

    cnt = lax.fori_loop(0, T // CH, chunk, jnp.zeros((1, NE), jnp.float32))

    tmf = jnp.float32(TM)
    padded = jnp.floor((cnt + (TM - 1)) / tmf) * tmf
    er = lax.broadcasted_iota(jnp.int32, (NE, NE), 0)
    ec = lax.broadcasted_iota(jnp.int32, (NE, NE), 1)
    t8 = (er < ec).astype(jnp.float32)
    offs = lax.dot_general(padded, t8, (((1,), (0,)), ((), ())),
                           preferred_element_type=jnp.float32)
    off0 = jnp.sum(jnp.where(col == e0, offs, jnp.float32(0.0)), axis=1,
                   keepdims=True)
    off1 = jnp.sum(jnp.where(col == e1, offs, jnp.float32(0.0)), axis=1,
                   keepdims=True)
    p0_ref[...] = (off0 + r0_ref[...]).astype(jnp.int32)
    p1_ref[...] = (off1 + r1_ref[...]).astype(jnp.int32)
    starts = offs / tmf
    tj = lax.broadcasted_iota(jnp.int32, (NT, NE), 0).astype(jnp.float32)
    te_ref[...] = (jnp.sum((tj >= starts).astype(jnp.float32), axis=1,
                           keepdims=True) - 1.0).astype(jnp.int32)


def _grouped_body(te_ref, xs_ref, w1_ref, w3_ref, w2_ref, out_ref,
                  xsb_ref, w1b_ref, w3b_ref, w2b_ref):
    f = pl.program_id(0)
    i = pl.program_id(1)
    sl = pl.ds(i * TM, TM)
    prev = te_ref[jnp.maximum(i - 1, 0), 0]
    changed = (i == 0) | (te_ref[i, 0] != prev)

    @pl.when(changed)
    def _cast():
        w1b_ref[...] = w1_ref[0].astype(jnp.bfloat16)
        w3b_ref[...] = w3_ref[0].astype(jnp.bfloat16)
        w2b_ref[...] = w2_ref[0].astype(jnp.bfloat16)

    @pl.when(f == 0)
    def _cx():
        xsb_ref[sl, :] = xs_ref[...].astype(jnp.bfloat16)

    xb = xsb_ref[sl, :]
    y1 = lax.dot_general(xb, w1b_ref[...], (((1,), (1,)), ((), ())),
                         preferred_element_type=jnp.float32)
    y3 = lax.dot_general(xb, w3b_ref[...], (((1,), (1,)), ((), ())),
                         preferred_element_type=jnp.float32)
    h = ((y1 * lax.logistic(y1)) * y3).astype(jnp.bfloat16)
    yp = lax.dot_general(h, w2b_ref[...], (((1,), (1,)), ((), ())),
                         preferred_element_type=jnp.float32)

    @pl.when(f == 0)
    def _set():
        out_ref[sl, :] = yp

    @pl.when(f != 0)
    def _acc():
        out_ref[sl, :] += yp


def _combine_body(a_ref, b_ref, w0_ref, w1_ref, o_ref):
    o_ref[...] = a_ref[...] * w0_ref[...] + b_ref[...] * w1_ref[...]


def _sc_mesh():
    return plsc.VectorSubcoreMesh(core_axis_name="c", subcore_axis_name="s")


def _dispatch_scatter(x_f32, pos0, pos1):
    @functools.partial(
        pl.kernel, mesh=_sc_mesh(),
        out_type=jax.ShapeDtypeStruct((NPAD, HIDDEN), jnp.float32),
        scratch_types=[
            pltpu.VMEM((TPW,), jnp.int32),
            pltpu.VMEM((TPW,), jnp.int32),
            pltpu.VMEM((TPW, HIDDEN), jnp.float32),
            pltpu.SemaphoreType.DMA,
        ],
    )
    def k(x_hbm, p0_hbm, p1_hbm, xs_hbm, i0_v, i1_v, rows_v, sem):
        wid = lax.axis_index("s") * 2 + lax.axis_index("c")
        base = wid * TPW
        pltpu.sync_copy(p0_hbm.at[pl.ds(base, TPW)], i0_v)
        pltpu.sync_copy(p1_hbm.at[pl.ds(base, TPW)], i1_v)
        pltpu.sync_copy(x_hbm.at[pl.ds(base, TPW)], rows_v)
        pltpu.async_copy(rows_v, xs_hbm.at[i0_v], sem).wait()
        pltpu.async_copy(rows_v, xs_hbm.at[i1_v], sem).wait()

    return k(x_f32, pos0, pos1)


def _combine_gather(ys, pos0, pos1):
    @functools.partial(
        pl.kernel, mesh=_sc_mesh(),
        out_type=(jax.ShapeDtypeStruct((T, HIDDEN), jnp.float32),
                  jax.ShapeDtypeStruct((T, HIDDEN), jnp.float32)),
        scratch_types=[
            pltpu.VMEM((TPW,), jnp.int32),
            pltpu.VMEM((TPW,), jnp.int32),
            pltpu.VMEM((TPW, HIDDEN), jnp.float32),
            pltpu.SemaphoreType.DMA,
        ],
    )
    def k(ys_hbm, p0_hbm, p1_hbm, a_hbm, b_hbm, i0_v, i1_v, rows_v, sem):
        wid = lax.axis_index("s") * 2 + lax.axis_index("c")
        base = wid * TPW
        pltpu.sync_copy(p0_hbm.at[pl.ds(base, TPW)], i0_v)
        pltpu.sync_copy(p1_hbm.at[pl.ds(base, TPW)], i1_v)
        pltpu.async_copy(ys_hbm.at[i0_v], rows_v, sem).wait()
        pltpu.sync_copy(rows_v, a_hbm.at[pl.ds(base, TPW)])
        pltpu.async_copy(ys_hbm.at[i1_v], rows_v, sem).wait()
        pltpu.sync_copy(rows_v, b_hbm.at[pl.ds(base, TPW)])

    return k(ys, pos0, pos1)


def kernel(hidden_states, gate_w, w1, w3, w2):
    b, s, hd = hidden_states.shape
    x2 = hidden_states.reshape(T, HIDDEN)

    (logits, wt0, wt1, pos0, pos1, tile_expert) = pl.pallas_call(
        _router_body,
        out_shape=(
            jax.ShapeDtypeStruct((T, NE), jnp.float32),
            jax.ShapeDtypeStruct((T, 1), jnp.float32),
            jax.ShapeDtypeStruct((T, 1), jnp.float32),
            jax.ShapeDtypeStruct((T, 1), jnp.int32),
            jax.ShapeDtypeStruct((T, 1), jnp.int32),
            jax.ShapeDtypeStruct((NT, 1), jnp.int32),
        ),
        scratch_shapes=[
            pltpu.VMEM((T, 1), jnp.int32),
            pltpu.VMEM((T, 1), jnp.int32),
            pltpu.VMEM((T, 1), jnp.float32),
            pltpu.VMEM((T, 1), jnp.float32),
            pltpu.VMEM((T, NE), jnp.float32),
        ],
    )(x2, gate_w)
    pos0 = pos0.reshape(T)
    pos1 = pos1.reshape(T)

    xs = _dispatch_scatter(x2, pos0, pos1)

    ys = pl.pallas_call(
        _grouped_body,
        grid_spec=pltpu.PrefetchScalarGridSpec(
            num_scalar_prefetch=1,
            grid=(NF, NT),
            in_specs=[
                pl.BlockSpec((TM, HIDDEN),
                             lambda f, i, te: (jnp.where(f == 0, i, 0), 0)),
                pl.BlockSpec((1, F_TILE, HIDDEN), lambda f, i, te: (te[i, 0], f, 0)),
                pl.BlockSpec((1, F_TILE, HIDDEN), lambda f, i, te: (te[i, 0], f, 0)),
                pl.BlockSpec((1, HIDDEN, F_TILE), lambda f, i, te: (te[i, 0], 0, f)),
            ],
            out_specs=pl.BlockSpec((NPAD, HIDDEN), lambda f, i, te: (0, 0)),
            scratch_shapes=[
                pltpu.VMEM((NPAD, HIDDEN), jnp.bfloat16),
                pltpu.VMEM((F_TILE, HIDDEN), jnp.bfloat16),
                pltpu.VMEM((F_TILE, HIDDEN), jnp.bfloat16),
                pltpu.VMEM((HIDDEN, F_TILE), jnp.bfloat16),
            ],
        ),
        out_shape=jax.ShapeDtypeStruct((NPAD, HIDDEN), jnp.float32),
    )(tile_expert, xs, w1, w3, w2)

    a, bb = _combine_gather(ys, pos0, pos1)

    out = pl.pallas_call(
        _combine_body,
        out_shape=jax.ShapeDtypeStruct((T, HIDDEN), jnp.float32),
    )(a, bb, wt0, wt1)

    return out.reshape(b, s, hd), logits
